# Initial kernel scaffold; baseline (speedup 1.0000x reference)
import jax
import jax.numpy as jnp
from jax.experimental import pallas as pl


def kernel(x, edge_index, W1, b1, W2, b2):
    raise NotImplementedError("write your pallas kernel here")


# speedup vs baseline: 30.8054x; 30.8054x over previous
import functools

import jax
import jax.numpy as jnp
from jax import lax
from jax.experimental import pallas as pl
from jax.experimental.pallas import tpu as pltpu
from jax.experimental.pallas import tpu_sc as plsc

N = 10000
E = 320000
D_IN = 128
D_HID = 16
D_OUT = 40

NC = 2
NS = 16
NW = NC * NS

B = 80
CH = E // B // NW
N_ACC = 10112

@functools.lru_cache(maxsize=None)
def _sc_kernels():
    mesh = plsc.VectorSubcoreMesh(
        core_axis_name="c", subcore_axis_name="s", num_cores=NC, num_subcores=NS
    )

    @functools.partial(
        pl.kernel,
        out_type=[
            jax.ShapeDtypeStruct((N_ACC,), jnp.float32),
            jax.ShapeDtypeStruct((N_ACC,), jnp.float32),
        ],
        mesh=mesh,
        scratch_types=[
            pltpu.VMEM((CH, B), jnp.int32),
            pltpu.VMEM((B,), jnp.float32),
            pltpu.VMEM((N_ACC // NS,), jnp.float32),
            pltpu.VMEM_SHARED((N_ACC,), jnp.float32),
        ],
    )
    def deg_kernel(dst_hbm, zero_hbm, out0_hbm, out1_hbm, dst_v, ones_v, slab_v, acc_sh):
        cid = lax.axis_index("c")
        sid = lax.axis_index("s")
        wid = sid * NC + cid

        @pl.when(sid == 0)
        def _():
            pltpu.sync_copy(zero_hbm, acc_sh)

        for i in range(B // 16):
            ones_v[pl.ds(i * 16, 16)] = jnp.full((16,), 1.0, jnp.float32)
        pltpu.sync_copy(dst_hbm.at[wid], dst_v)
        plsc.subcore_barrier()

        def body(j, _):
            pltpu.sync_copy(ones_v, acc_sh.at[dst_v.at[j]], add=True)
            return 0

        lax.fori_loop(0, CH, body, 0)
        plsc.subcore_barrier()

        rpt = N_ACC // NS
        pltpu.sync_copy(acc_sh.at[pl.ds(sid * rpt, rpt)], slab_v)

        @pl.when(cid == 0)
        def _():
            pltpu.sync_copy(slab_v, out0_hbm.at[pl.ds(sid * rpt, rpt)])

        @pl.when(cid == 1)
        def _():
            pltpu.sync_copy(slab_v, out1_hbm.at[pl.ds(sid * rpt, rpt)])

    @functools.partial(
        pl.kernel,
        out_type=jax.ShapeDtypeStruct((NC, N_ACC, D_HID), jnp.float32),
        mesh=mesh,
        scratch_types=[
            pltpu.VMEM((CH, B), jnp.int32),
            pltpu.VMEM((CH, B), jnp.int32),
            pltpu.VMEM((B, D_HID), jnp.float32),
            pltpu.VMEM((N_ACC // NS, D_HID), jnp.float32),
            pltpu.VMEM_SHARED((N_ACC, D_HID), jnp.float32),
        ],
        compiler_params=pltpu.CompilerParams(use_tc_tiling_on_sc=False),
    )
    def agg_kernel(src_hbm, dst_hbm, g_hbm, zero_hbm, out_hbm, src_v, dst_v, rows_v, slab_v, acc_sh):
        cid = lax.axis_index("c")
        sid = lax.axis_index("s")
        wid = sid * NC + cid

        @pl.when(sid == 0)
        def _():
            pltpu.sync_copy(zero_hbm, acc_sh)

        pltpu.sync_copy(src_hbm.at[wid], src_v)
        pltpu.sync_copy(dst_hbm.at[wid], dst_v)
        plsc.subcore_barrier()

        def body(j, _):
            pltpu.sync_copy(g_hbm.at[src_v.at[j]], rows_v)
            pltpu.sync_copy(rows_v, acc_sh.at[dst_v.at[j]], add=True)
            return 0

        lax.fori_loop(0, CH, body, 0)
        plsc.subcore_barrier()

        rpt = N_ACC // NS
        pltpu.sync_copy(acc_sh.at[pl.ds(sid * rpt, rpt)], slab_v)
        pltpu.sync_copy(slab_v, out_hbm.at[cid, pl.ds(sid * rpt, rpt)])

    return deg_kernel, agg_kernel


def _mm1_body(x_ref, w_ref, p0_ref, p1_ref, g_ref, dinv_ref):
    deg = p0_ref[:N] + p1_ref[:N] + 1.0
    dinv = lax.rsqrt(deg)
    h = jnp.dot(x_ref[...], w_ref[...], preferred_element_type=jnp.float32)
    g_ref[...] = h * dinv[:, None]
    dinv_ref[...] = dinv


def _relu_body(q_ref, g1_ref, dinv_ref, b_ref, g2_ref):
    dinv = dinv_ref[...][:, None]
    s = (q_ref[0, :N, :] + q_ref[1, :N, :] + g1_ref[...]) * dinv
    h = jnp.maximum(s + b_ref[...][None, :], 0.0)
    g2_ref[...] = h * dinv


def _mm2_body(r_ref, g2_ref, dinv_ref, w_ref, b_ref, out_ref):
    a = (r_ref[0, :N, :] + r_ref[1, :N, :] + g2_ref[...]) * dinv_ref[...][:, None]
    out_ref[...] = (
        jnp.dot(a, w_ref[...], preferred_element_type=jnp.float32)
        + b_ref[...][None, :]
    )


def kernel(x, edge_index, W1, b1, W2, b2):
    src = edge_index[0].astype(jnp.int32).reshape(NW, CH, B)
    dst = edge_index[1].astype(jnp.int32).reshape(NW, CH, B)
    z1 = jnp.zeros((N_ACC,), jnp.float32)
    z16 = jnp.zeros((N_ACC, D_HID), jnp.float32)
    _deg_kernel, _agg_kernel = _sc_kernels()

    p0, p1 = _deg_kernel(dst, z1)

    g1, dinv = pl.pallas_call(
        _mm1_body,
        out_shape=[
            jax.ShapeDtypeStruct((N, D_HID), jnp.float32),
            jax.ShapeDtypeStruct((N,), jnp.float32),
        ],
    )(x, W1, p0, p1)

    q = _agg_kernel(src, dst, g1, z16)

    g2 = pl.pallas_call(
        _relu_body,
        out_shape=jax.ShapeDtypeStruct((N, D_HID), jnp.float32),
    )(q, g1, dinv, b1)

    r = _agg_kernel(src, dst, g2, z16)

    out = pl.pallas_call(
        _mm2_body,
        out_shape=jax.ShapeDtypeStruct((N, D_OUT), jnp.float32),
    )(r, g2, dinv, W2, b2)
    return out


# speedup vs baseline: 62.2884x; 2.0220x over previous
import functools

import jax
import jax.numpy as jnp
from jax import lax
from jax.experimental import pallas as pl
from jax.experimental.pallas import tpu as pltpu
from jax.experimental.pallas import tpu_sc as plsc

N = 10000
E = 320000
D_IN = 128
D_HID = 16
D_OUT = 40

NC = 2
NS = 16
NW = NC * NS

B = 80
CH = E // B // NW
N_ACC = 10112
NBUF = 5

@functools.lru_cache(maxsize=None)
def _sc_kernels():
    mesh = plsc.VectorSubcoreMesh(
        core_axis_name="c", subcore_axis_name="s", num_cores=NC, num_subcores=NS
    )

    @functools.partial(
        pl.kernel,
        out_type=[
            jax.ShapeDtypeStruct((N_ACC,), jnp.float32),
            jax.ShapeDtypeStruct((N_ACC,), jnp.float32),
        ],
        mesh=mesh,
        scratch_types=[
            pltpu.VMEM((CH, B), jnp.int32),
            pltpu.VMEM((B,), jnp.float32),
            pltpu.VMEM((N_ACC // NS,), jnp.float32),
            pltpu.VMEM_SHARED((N_ACC,), jnp.float32),
            pltpu.SemaphoreType.DMA,
        ],
    )
    def deg_kernel(dst_hbm, zero_hbm, out0_hbm, out1_hbm, dst_v, ones_v, slab_v, acc_sh, sem):
        cid = lax.axis_index("c")
        sid = lax.axis_index("s")
        wid = sid * NC + cid

        @pl.when(sid == 0)
        def _():
            pltpu.sync_copy(zero_hbm, acc_sh)

        for i in range(B // 16):
            ones_v[pl.ds(i * 16, 16)] = jnp.full((16,), 1.0, jnp.float32)
        pltpu.sync_copy(dst_hbm.at[wid], dst_v)
        plsc.subcore_barrier()

        def body(j, _):
            pltpu.async_copy(ones_v, acc_sh.at[dst_v.at[j]], sem, add=True)
            return 0

        lax.fori_loop(0, CH, body, 0)

        def drain(j, _):
            pltpu.make_async_copy(ones_v, acc_sh.at[dst_v.at[j]], sem).wait()
            return 0

        lax.fori_loop(0, CH, drain, 0)
        plsc.subcore_barrier()

        rpt = N_ACC // NS
        pltpu.sync_copy(acc_sh.at[pl.ds(sid * rpt, rpt)], slab_v)

        @pl.when(cid == 0)
        def _():
            pltpu.sync_copy(slab_v, out0_hbm.at[pl.ds(sid * rpt, rpt)])

        @pl.when(cid == 1)
        def _():
            pltpu.sync_copy(slab_v, out1_hbm.at[pl.ds(sid * rpt, rpt)])

    @functools.partial(
        pl.kernel,
        out_type=jax.ShapeDtypeStruct((NC, N_ACC, D_HID), jnp.float32),
        mesh=mesh,
        scratch_types=[
            pltpu.VMEM((CH, B), jnp.int32),
            pltpu.VMEM((CH, B), jnp.int32),
            pltpu.VMEM((NBUF, B, D_HID), jnp.float32),
            pltpu.VMEM((N_ACC // NS, D_HID), jnp.float32),
            pltpu.VMEM_SHARED((N_ACC, D_HID), jnp.float32),
        ]
        + [pltpu.SemaphoreType.DMA] * NBUF,
        compiler_params=pltpu.CompilerParams(use_tc_tiling_on_sc=False),
    )
    def agg_kernel(
        src_hbm, dst_hbm, g_hbm, zero_hbm, out_hbm, src_v, dst_v, rows_v, slab_v, acc_sh, *sems
    ):
        cid = lax.axis_index("c")
        sid = lax.axis_index("s")
        wid = sid * NC + cid

        @pl.when(sid == 0)
        def _():
            pltpu.sync_copy(zero_hbm, acc_sh)

        pltpu.sync_copy(src_hbm.at[wid], src_v)
        pltpu.sync_copy(dst_hbm.at[wid], dst_v)
        plsc.subcore_barrier()

        for b in range(NBUF):
            pltpu.async_copy(g_hbm.at[src_v.at[b]], rows_v.at[b], sems[b])

        def group(gi, _):
            for b in range(NBUF):
                j = gi * NBUF + b
                pltpu.make_async_copy(g_hbm.at[src_v.at[j]], rows_v.at[b], sems[b]).wait()
                pltpu.sync_copy(rows_v.at[b], acc_sh.at[dst_v.at[j]], add=True)
                jn = j + NBUF

                @pl.when(jn < CH)
                def _():
                    pltpu.async_copy(g_hbm.at[src_v.at[jn]], rows_v.at[b], sems[b])
            return 0

        lax.fori_loop(0, CH // NBUF, group, 0)
        plsc.subcore_barrier()

        rpt = N_ACC // NS
        pltpu.sync_copy(acc_sh.at[pl.ds(sid * rpt, rpt)], slab_v)
        pltpu.sync_copy(slab_v, out_hbm.at[cid, pl.ds(sid * rpt, rpt)])

    return deg_kernel, agg_kernel


def _mm1_body(x_ref, w_ref, p0_ref, p1_ref, g_ref, dinv_ref):
    deg = p0_ref[:N] + p1_ref[:N] + 1.0
    dinv = lax.rsqrt(deg)
    h = jnp.dot(x_ref[...], w_ref[...], preferred_element_type=jnp.float32)
    g_ref[...] = h * dinv[:, None]
    dinv_ref[...] = dinv


def _relu_body(q_ref, g1_ref, dinv_ref, b_ref, g2_ref):
    dinv = dinv_ref[...][:, None]
    s = (q_ref[0, :N, :] + q_ref[1, :N, :] + g1_ref[...]) * dinv
    h = jnp.maximum(s + b_ref[...][None, :], 0.0)
    g2_ref[...] = h * dinv


def _mm2_body(r_ref, g2_ref, dinv_ref, w_ref, b_ref, out_ref):
    a = (r_ref[0, :N, :] + r_ref[1, :N, :] + g2_ref[...]) * dinv_ref[...][:, None]
    out_ref[...] = (
        jnp.dot(a, w_ref[...], preferred_element_type=jnp.float32)
        + b_ref[...][None, :]
    )


def kernel(x, edge_index, W1, b1, W2, b2):
    src = edge_index[0].astype(jnp.int32).reshape(NW, CH, B)
    dst = edge_index[1].astype(jnp.int32).reshape(NW, CH, B)
    z1 = jnp.zeros((N_ACC,), jnp.float32)
    z16 = jnp.zeros((N_ACC, D_HID), jnp.float32)
    _deg_kernel, _agg_kernel = _sc_kernels()

    p0, p1 = _deg_kernel(dst, z1)

    g1, dinv = pl.pallas_call(
        _mm1_body,
        out_shape=[
            jax.ShapeDtypeStruct((N, D_HID), jnp.float32),
            jax.ShapeDtypeStruct((N,), jnp.float32),
        ],
    )(x, W1, p0, p1)

    q = _agg_kernel(src, dst, g1, z16)

    g2 = pl.pallas_call(
        _relu_body,
        out_shape=jax.ShapeDtypeStruct((N, D_HID), jnp.float32),
    )(q, g1, dinv, b1)

    r = _agg_kernel(src, dst, g2, z16)

    out = pl.pallas_call(
        _mm2_body,
        out_shape=jax.ShapeDtypeStruct((N, D_OUT), jnp.float32),
    )(r, g2, dinv, W2, b2)
    return out
